# baseline (speedup 1.0000x reference)
import functools

import jax
import jax.numpy as jnp
from jax import lax
from jax.experimental import pallas as pl
from jax.experimental.pallas import tpu as pltpu
from jax.experimental.pallas import tpu_sc as plsc

B = 16384
V = 1000
D = 16
N_FIELDS = 3
NC = 2
NS = 16
NW = NC * NS
CHUNK = B // NW
GROUPS = CHUNK // 16
WQ_OFF = 8
WQ_LEN = 3072
B_OFF = WQ_OFF + 3002

_MESH = plsc.VectorSubcoreMesh(core_axis_name="c", subcore_axis_name="s")


@functools.partial(
    pl.kernel,
    mesh=_MESH,
    compiler_params=pltpu.CompilerParams(needs_layout_passes=False),
    out_type=jax.ShapeDtypeStruct((B,), jnp.float32),
    scratch_types=[
        pltpu.VMEM((CHUNK,), jnp.int32),
        pltpu.VMEM((CHUNK,), jnp.float32),
        pltpu.VMEM((CHUNK,), jnp.float32),
        pltpu.VMEM((N_FIELDS * V * 17 + 72,), jnp.float32),
        pltpu.VMEM((WQ_LEN,), jnp.float32),
        pltpu.VMEM((CHUNK,), jnp.float32),
        pltpu.SemaphoreType.DMA,
    ],
)
def _fm_forward(packed, den0, den1, tflat, wq, out,
                iv, dv0, dv1, tv, wv, outv, sem):
    wid = lax.axis_index("s") * NC + lax.axis_index("c")
    base = wid * CHUNK

    cpa = pltpu.async_copy(tflat, tv, sem)
    cpb = pltpu.async_copy(wq, wv, sem)
    cpc = pltpu.async_copy(packed.at[pl.ds(base, CHUNK)], iv, sem)
    cpd = pltpu.async_copy(den0.at[pl.ds(base, CHUNK)], dv0, sem)
    cpe = pltpu.async_copy(den1.at[pl.ds(base, CHUNK)], dv1, sem)
    cpa.wait()
    cpb.wait()
    cpc.wait()
    cpd.wait()
    cpe.wait()

    zero16 = jnp.zeros((16,), jnp.int32)
    w0v = plsc.load_gather(wv, [zero16 + WQ_OFF])
    w1v = plsc.load_gather(wv, [zero16 + (WQ_OFF + 1)])
    bv = plsc.load_gather(wv, [zero16 + B_OFF])

    UNROLL = 4

    def one_group(off):
        pk = iv[pl.ds(off, 16)]
        i0 = pk & 1023
        i1 = (pk >> 10) & 1023
        i2 = pk >> 20
        d0 = dv0[pl.ds(off, 16)]
        d1 = dv1[pl.ds(off, 16)]
        lw = (plsc.load_gather(wv, [i0 + (WQ_OFF + 2)])
              + plsc.load_gather(wv, [i1 + (WQ_OFF + 2 + V)])
              + plsc.load_gather(wv, [i2 + (WQ_OFF + 2 + 2 * V)]))
        lin = d0 * w0v + d1 * w1v + bv + lw
        ib0 = i0 * 17
        ib1 = i1 * 17 + V * 17
        ib2 = i2 * 17 + 2 * V * 17
        accs = [lin, None, None, None]
        for d in range(D):
            a0 = plsc.load_gather(tv, [ib0 + d])
            a1 = plsc.load_gather(tv, [ib1 + d])
            a2 = plsc.load_gather(tv, [ib2 + d])
            term = a0 * a1 + a2 * (a0 + a1)
            k = d % 4
            accs[k] = term if accs[k] is None else accs[k] + term
        acc = (accs[0] + accs[1]) + (accs[2] + accs[3])
        outv[pl.ds(off, 16)] = 1.0 / (1.0 + jnp.exp(-acc))

    def body(g, carry):
        base_off = g * (16 * UNROLL)
        for u in range(UNROLL):
            one_group(base_off + u * 16)
        return carry

    lax.fori_loop(0, GROUPS // UNROLL, body, 0)
    pltpu.sync_copy(outv, out.at[pl.ds(base, CHUNK)])


def kernel(inputs, tables, w, b):
    ids = inputs[:, :N_FIELDS].astype(jnp.int32)
    packed = ids[:, 0] | (ids[:, 1] << 10) | (ids[:, 2] << 20)
    wq = jnp.concatenate(
        [jnp.zeros((WQ_OFF,), jnp.float32), w.reshape(-1), b,
         jnp.zeros((WQ_LEN - B_OFF - 1,), jnp.float32)])
    tpad = jnp.concatenate(
        [tables.reshape(N_FIELDS * V, D),
         jnp.zeros((N_FIELDS * V, 1), jnp.float32)], axis=1).reshape(-1)
    tpad = jnp.concatenate([tpad, jnp.zeros((72,), jnp.float32)])
    out = _fm_forward(packed, inputs[:, N_FIELDS], inputs[:, N_FIELDS + 1],
                      tpad, wq)
    return out.reshape(B, 1)


# speedup vs baseline: 1.0698x; 1.0698x over previous
import functools

import jax
import jax.numpy as jnp
from jax import lax
from jax.experimental import pallas as pl
from jax.experimental.pallas import tpu as pltpu
from jax.experimental.pallas import tpu_sc as plsc

B = 16384
V = 1000
D = 16
N_FIELDS = 3
NC = 2
NS = 16
NW = NC * NS
CHUNK = B // NW
GROUPS = CHUNK // 16
WQ_OFF = 8
WQ_LEN = 3072
B_OFF = WQ_OFF + 3002

_MESH = plsc.VectorSubcoreMesh(core_axis_name="c", subcore_axis_name="s")


@functools.partial(
    pl.kernel,
    mesh=_MESH,
    compiler_params=pltpu.CompilerParams(needs_layout_passes=False),
    out_type=jax.ShapeDtypeStruct((B,), jnp.float32),
    scratch_types=[
        pltpu.VMEM((CHUNK,), jnp.int32),
        pltpu.VMEM((CHUNK,), jnp.float32),
        pltpu.VMEM((CHUNK,), jnp.float32),
        pltpu.VMEM((N_FIELDS * V * D,), jnp.float32),
        pltpu.VMEM((WQ_LEN,), jnp.float32),
        pltpu.VMEM((CHUNK,), jnp.float32),
        pltpu.SemaphoreType.DMA,
    ],
)
def _fm_forward(packed, den0, den1, tflat, wq, out,
                iv, dv0, dv1, tv, wv, outv, sem):
    wid = lax.axis_index("s") * NC + lax.axis_index("c")
    base = wid * CHUNK

    cpa = pltpu.async_copy(tflat, tv, sem)
    cpb = pltpu.async_copy(wq, wv, sem)
    cpc = pltpu.async_copy(packed.at[pl.ds(base, CHUNK)], iv, sem)
    cpd = pltpu.async_copy(den0.at[pl.ds(base, CHUNK)], dv0, sem)
    cpe = pltpu.async_copy(den1.at[pl.ds(base, CHUNK)], dv1, sem)
    cpa.wait()
    cpb.wait()
    cpc.wait()
    cpd.wait()
    cpe.wait()

    zero16 = jnp.zeros((16,), jnp.int32)
    w0v = plsc.load_gather(wv, [zero16 + WQ_OFF])
    w1v = plsc.load_gather(wv, [zero16 + (WQ_OFF + 1)])
    bv = plsc.load_gather(wv, [zero16 + B_OFF])

    UNROLL = 4

    def one_group(off):
        pk = iv[pl.ds(off, 16)]
        i0 = pk & 1023
        i1 = (pk >> 10) & 1023
        i2 = pk >> 20
        d0 = dv0[pl.ds(off, 16)]
        d1 = dv1[pl.ds(off, 16)]
        lw = (plsc.load_gather(wv, [i0 + (WQ_OFF + 2)])
              + plsc.load_gather(wv, [i1 + (WQ_OFF + 2 + V)])
              + plsc.load_gather(wv, [i2 + (WQ_OFF + 2 + 2 * V)]))
        lin = d0 * w0v + d1 * w1v + bv + lw
        ib0 = i0
        ib1 = i1 + V
        ib2 = i2 + 2 * V
        accs = [lin, None, None, None]
        for d in range(D):
            a0 = plsc.load_gather(tv, [ib0 + d * (N_FIELDS * V)])
            a1 = plsc.load_gather(tv, [ib1 + d * (N_FIELDS * V)])
            a2 = plsc.load_gather(tv, [ib2 + d * (N_FIELDS * V)])
            term = a0 * a1 + a2 * (a0 + a1)
            k = d % 4
            accs[k] = term if accs[k] is None else accs[k] + term
        acc = (accs[0] + accs[1]) + (accs[2] + accs[3])
        outv[pl.ds(off, 16)] = 1.0 / (1.0 + jnp.exp(-acc))

    def body(g, carry):
        base_off = g * (16 * UNROLL)
        for u in range(UNROLL):
            one_group(base_off + u * 16)
        return carry

    lax.fori_loop(0, GROUPS // UNROLL, body, 0)
    pltpu.sync_copy(outv, out.at[pl.ds(base, CHUNK)])


def kernel(inputs, tables, w, b):
    ids = inputs[:, :N_FIELDS].astype(jnp.int32)
    packed = ids[:, 0] | (ids[:, 1] << 10) | (ids[:, 2] << 20)
    wq = jnp.concatenate(
        [jnp.zeros((WQ_OFF,), jnp.float32), w.reshape(-1), b,
         jnp.zeros((WQ_LEN - B_OFF - 1,), jnp.float32)])
    tmaj = tables.reshape(N_FIELDS * V, D).T.reshape(-1)
    out = _fm_forward(packed, inputs[:, N_FIELDS], inputs[:, N_FIELDS + 1],
                      tmaj, wq)
    return out.reshape(B, 1)
